# baseline (speedup 1.0000x reference)
import functools

import jax
import jax.numpy as jnp
from jax import lax
from jax.experimental import pallas as pl
from jax.experimental.pallas import tpu as pltpu
from jax.experimental.pallas import tpu_sc as plsc

NC = 2
NS = 16
NW = NC * NS


def _sc_body(n_cls, n_ctx, s, d, base_cnt, rem,
             ctx_hbm, pre_hbm, suf_hbm, out_hbm,
             rep_v, sem_pre, sem_suf, sem_ctx):
    cid = lax.axis_index("c")
    sid = lax.axis_index("s")
    wid = sid * NC + cid
    base = wid * base_cnt + jnp.minimum(wid, rem)
    has_extra = wid < rem

    pltpu.sync_copy(ctx_hbm, rep_v)

    pre_cp = pltpu.make_async_copy(
        pre_hbm.at[pl.ds(base, base_cnt)],
        out_hbm.at[pl.ds(base, base_cnt), pl.ds(0, 1)], sem_pre)
    pre_cp.start()

    suf_cp = pltpu.make_async_copy(
        suf_hbm.at[pl.ds(base, base_cnt)],
        out_hbm.at[pl.ds(base, base_cnt), pl.ds(1 + n_ctx, s)], sem_suf)
    suf_cp.start()

    for i in range(base_cnt):
        pltpu.make_async_copy(
            rep_v, out_hbm.at[pl.ds(base + i, 1), pl.ds(1, n_ctx)],
            sem_ctx).start()

    @pl.when(has_extra)
    def _():
        pltpu.make_async_copy(
            pre_hbm.at[pl.ds(base + base_cnt, 1)],
            out_hbm.at[pl.ds(base + base_cnt, 1), pl.ds(0, 1)], sem_pre).start()
        pltpu.make_async_copy(
            suf_hbm.at[pl.ds(base + base_cnt, 1)],
            out_hbm.at[pl.ds(base + base_cnt, 1), pl.ds(1 + n_ctx, s)],
            sem_suf).start()
        pltpu.make_async_copy(
            rep_v,
            out_hbm.at[pl.ds(base + base_cnt, 1), pl.ds(1, n_ctx)],
            sem_ctx).start()

    pre_cp.wait()
    suf_cp.wait()
    for i in range(base_cnt):
        pltpu.make_async_copy(
            rep_v, out_hbm.at[pl.ds(base, 1), pl.ds(1, n_ctx)], sem_ctx).wait()

    @pl.when(has_extra)
    def _():
        pltpu.make_async_copy(
            pre_hbm.at[pl.ds(base, 1)],
            out_hbm.at[pl.ds(base, 1), pl.ds(0, 1)], sem_pre).wait()
        pltpu.make_async_copy(
            suf_hbm.at[pl.ds(base, 1)],
            out_hbm.at[pl.ds(base, 1), pl.ds(1 + n_ctx, s)], sem_suf).wait()
        pltpu.make_async_copy(
            rep_v, out_hbm.at[pl.ds(base, 1), pl.ds(1, n_ctx)], sem_ctx).wait()


def kernel(ctx, token_prefix, token_suffix):
    n_cls, _, d = token_prefix.shape
    n_ctx = ctx.shape[0]
    s = token_suffix.shape[1]
    seq = 1 + n_ctx + s
    base_cnt = n_cls // NW
    rem = n_cls - base_cnt * NW

    ctx3 = ctx.reshape(1, n_ctx, d)
    mesh = plsc.VectorSubcoreMesh(core_axis_name="c", subcore_axis_name="s")

    sck = functools.partial(
        pl.kernel,
        out_type=jax.ShapeDtypeStruct((n_cls, seq, d), jnp.float32),
        mesh=mesh,
        compiler_params=pltpu.CompilerParams(use_tc_tiling_on_sc=False),
        scratch_types=[
            pltpu.VMEM((1, n_ctx, d), jnp.float32),
            pltpu.SemaphoreType.DMA,
            pltpu.SemaphoreType.DMA,
            pltpu.SemaphoreType.DMA,
        ],
    )(functools.partial(_sc_body, n_cls, n_ctx, s, d, base_cnt, rem))

    return sck(ctx3, token_prefix, token_suffix)


# speedup vs baseline: 5.8748x; 5.8748x over previous
import functools

import jax
import jax.numpy as jnp
from jax import lax
from jax.experimental import pallas as pl
from jax.experimental.pallas import tpu as pltpu
from jax.experimental.pallas import tpu_sc as plsc

NC = 2
CHUNK = 20


def _sc_body(n_cls, n_ctx, s, d, per_core,
             ctx_hbm, pre_hbm, suf_hbm, out_hbm,
             rep_sp, pre_sp, suf_sp, sem_rep, sem_pre, sem_ctx, sem_si, sem_so):
    cid = lax.axis_index("c")
    base = cid * per_core
    n_chunks = per_core // CHUNK

    for r in range(CHUNK):
        pltpu.make_async_copy(ctx_hbm, rep_sp.at[pl.ds(r, 1)], sem_rep).start()

    pre_in = pltpu.make_async_copy(
        pre_hbm.at[pl.ds(base, per_core)], pre_sp, sem_pre)
    pre_in.start()

    def suf_in(c, buf):
        return pltpu.make_async_copy(
            suf_hbm.at[pl.ds(base + c * CHUNK, CHUNK)],
            suf_sp.at[pl.ds(buf * CHUNK, CHUNK)], sem_si)

    def suf_out(c, buf):
        return pltpu.make_async_copy(
            suf_sp.at[pl.ds(buf * CHUNK, CHUNK)],
            out_hbm.at[pl.ds(base + c * CHUNK, CHUNK), pl.ds(1 + n_ctx, s)],
            sem_so)

    suf_in(0, 0).start()
    for r in range(CHUNK):
        pltpu.make_async_copy(ctx_hbm, rep_sp.at[pl.ds(0, 1)], sem_rep).wait()

    for c in range(n_chunks):
        buf = c % 2
        suf_in(c, buf).wait()
        if c >= 1:
            suf_out(c - 1, 1 - buf).wait()
        if c + 1 < n_chunks:
            suf_in(c + 1, 1 - buf).start()
        suf_out(c, buf).start()
        pltpu.make_async_copy(
            rep_sp,
            out_hbm.at[pl.ds(base + c * CHUNK, CHUNK), pl.ds(1, n_ctx)],
            sem_ctx).start()

    pre_in.wait()
    pre_out = pltpu.make_async_copy(
        pre_sp, out_hbm.at[pl.ds(base, per_core), pl.ds(0, 1)], sem_pre)
    pre_out.start()

    suf_out(n_chunks - 1, (n_chunks - 1) % 2).wait()
    for c in range(n_chunks):
        pltpu.make_async_copy(
            rep_sp,
            out_hbm.at[pl.ds(base, CHUNK), pl.ds(1, n_ctx)], sem_ctx).wait()
    pre_out.wait()


def kernel(ctx, token_prefix, token_suffix):
    n_cls, _, d = token_prefix.shape
    n_ctx = ctx.shape[0]
    s = token_suffix.shape[1]
    seq = 1 + n_ctx + s
    per_core = n_cls // NC

    ctx3 = ctx.reshape(1, n_ctx, d)
    mesh = plsc.ScalarSubcoreMesh(axis_name="c", num_cores=NC)

    sck = functools.partial(
        pl.kernel,
        out_type=jax.ShapeDtypeStruct((n_cls, seq, d), jnp.float32),
        mesh=mesh,
        compiler_params=pltpu.CompilerParams(use_tc_tiling_on_sc=False),
        scratch_types=[
            pltpu.VMEM_SHARED((CHUNK, n_ctx, d), jnp.float32),
            pltpu.VMEM_SHARED((per_core, 1, d), jnp.float32),
            pltpu.VMEM_SHARED((2 * CHUNK, s, d), jnp.float32),
            pltpu.SemaphoreType.DMA,
            pltpu.SemaphoreType.DMA,
            pltpu.SemaphoreType.DMA,
            pltpu.SemaphoreType.DMA,
            pltpu.SemaphoreType.DMA,
        ],
    )(functools.partial(_sc_body, n_cls, n_ctx, s, d, per_core))

    return sck(ctx3, token_prefix, token_suffix)


# speedup vs baseline: 26.4913x; 4.5093x over previous
import jax
import jax.numpy as jnp
from jax.experimental import pallas as pl


def _body(pre_ref, ctx_ref, suf_ref, out_ref):
    b, _, d = pre_ref.shape
    n_ctx = ctx_ref.shape[0]
    s = suf_ref.shape[1]
    out_ref[:, 0:1, :] = pre_ref[...]
    out_ref[:, 1:1 + n_ctx, :] = jnp.broadcast_to(ctx_ref[...][None], (b, n_ctx, d))
    out_ref[:, 1 + n_ctx:, :] = suf_ref[...]


def kernel(ctx, token_prefix, token_suffix):
    n_cls, _, d = token_prefix.shape
    n_ctx = ctx.shape[0]
    s = token_suffix.shape[1]
    seq = 1 + n_ctx + s

    B = 50
    return pl.pallas_call(
        _body,
        grid=(n_cls // B,),
        in_specs=[
            pl.BlockSpec((B, 1, d), lambda i: (i, 0, 0)),
            pl.BlockSpec((n_ctx, d), lambda i: (0, 0)),
            pl.BlockSpec((B, s, d), lambda i: (i, 0, 0)),
        ],
        out_specs=pl.BlockSpec((B, seq, d), lambda i: (i, 0, 0)),
        out_shape=jax.ShapeDtypeStruct((n_cls, seq, d), jnp.float32),
    )(token_prefix, ctx, token_suffix)
